# baseline (speedup 1.0000x reference)
import functools

import jax
import jax.numpy as jnp
from jax import lax
from jax.experimental import pallas as pl
from jax.experimental.pallas import tpu as pltpu
from jax.experimental.pallas import tpu_sc as plsc

BS = 64
ENT_N = 16
SPO_N = 16
NUM_BOX = 1024
NUM_CTX = 5
LANES = 16
NCHUNK = NUM_BOX // LANES
UNROLL = 2
NW = 32
BPT = BS // NW
ROW_W = NUM_BOX * NUM_CTX
ENT_W = ENT_N * NUM_BOX
EPS = 1e-6


def _tdp_body(tl_hbm, adj_hbm, um_hbm, ent_hbm, spo_hbm, w_hbm, out_hbm,
              tl_v, adj_v, um_v, rowid_v, ring_v, ent_v, w_v, add_v,
              sem_meta, sem_ent, sem_w, sem_ring, sem_out):
    cid = lax.axis_index("c")
    sid = lax.axis_index("s")
    wid = sid * 2 + cid
    b0 = wid * BPT

    iota = lax.iota(jnp.int32, LANES)
    zeros_i = jnp.zeros((LANES,), jnp.int32)
    mask_ctx = iota < NUM_CTX
    eps_v = jnp.full((LANES,), EPS, jnp.float32)

    c_meta = []
    for j in range(BPT):
        c_meta.append(pltpu.async_copy(
            tl_hbm.at[b0 + j], tl_v.at[pl.ds(j * ENT_N, ENT_N)], sem_meta.at[0]))
        c_meta.append(pltpu.async_copy(
            adj_hbm.at[b0 + j], adj_v.at[pl.ds(j * ENT_N, ENT_N)], sem_meta.at[1]))
        c_meta.append(pltpu.async_copy(
            um_hbm.at[b0 + j], um_v.at[pl.ds(j * LANES, LANES)], sem_meta.at[2]))
    c_ent = [pltpu.async_copy(ent_hbm.at[b0 + j],
                              ent_v.at[pl.ds(j * ENT_W, ENT_W)], sem_ent.at[j])
             for j in range(BPT)]
    c_w = [pltpu.async_copy(w_hbm.at[b0 + j],
                            w_v.at[pl.ds(j * ENT_W, ENT_W)], sem_w.at[j])
           for j in range(BPT)]
    for c in c_meta:
        c.wait()

    for j in range(BPT):
        tlv = tl_v[pl.ds(j * ENT_N, ENT_N)]
        e0 = plsc.load_gather(adj_v, [j * ENT_N + tlv])
        rid = e0 + (b0 + j) * ENT_N
        plsc.store_scatter(rowid_v, [j * ENT_N + iota, zeros_i], rid)

    def _init(k, carry):
        plsc.store_scatter(add_v, [k * LANES + iota], eps_v)
        return carry
    lax.fori_loop(0, BPT * NCHUNK, _init, 0)

    handles = {}
    for pit in range(3):
        for j in range(BPT):
            ps = 2 * (pit % 4) + j
            handles[(j, pit)] = pltpu.async_copy(
                spo_hbm.at[rowid_v.at[j * ENT_N + pit]], ring_v.at[pl.ds(ps, 1)],
                sem_ring.at[ps])

    for j in range(BPT):
        c_ent[j].wait()
        c_w[j].wait()
    tlv = [tl_v[pl.ds(j * ENT_N, ENT_N)] for j in range(BPT)]
    umv = [um_v[pl.ds(j * LANES, LANES)] for j in range(BPT)]

    c_out = []
    for it in range(ENT_N):
        slots = [2 * (it % 4) + j for j in range(BPT)]
        if it + 3 < ENT_N:
            for j in range(BPT):
                ns = 2 * ((it + 3) % 4) + j
                handles[(j, it + 3)] = pltpu.async_copy(
                    spo_hbm.at[rowid_v.at[j * ENT_N + it + 3]],
                    ring_v.at[pl.ds(ns, 1)], sem_ring.at[ns])
        for j in range(BPT):
            handles[(j, it)].wait()

        p_spl = [jnp.full((LANES,), tlv[j][it], jnp.int32) for j in range(BPT)]
        pbase = [p_spl[j] * NUM_BOX + (j * ENT_W) for j in range(BPT)]
        rslot = [jnp.full((LANES,), slots[j], jnp.int32) for j in range(BPT)]

        def _dot(k, accs):
            accs = list(accs)
            for u in range(UNROLL):
                kvec = (k * UNROLL + u) * LANES + iota
                base5 = kvec * NUM_CTX
                for j in range(BPT):
                    e0c = plsc.load_gather(ent_v, [j * ENT_W + kvec])
                    for c in range(NUM_CTX):
                        accs[j * NUM_CTX + c] = accs[j * NUM_CTX + c] + e0c * \
                            plsc.load_gather(ring_v, [rslot[j], base5 + c])
            return tuple(accs)
        accs = lax.fori_loop(
            0, NCHUNK // UNROLL, _dot,
            tuple(jnp.zeros((LANES,), jnp.float32) for _ in range(BPT * NUM_CTX)))
        for j in range(BPT):
            tvec = jnp.zeros((LANES,), jnp.float32)
            for c in range(NUM_CTX):
                tvec = jnp.where(iota == c,
                                 jnp.sum(accs[j * NUM_CTX + c]) + jnp.float32(EPS),
                                 tvec)
            plsc.store_scatter(add_v, [j * NUM_BOX + umv[j]], tvec, mask=mask_ctx)

        def _upd(k, mx):
            mx = list(mx)
            for u in range(UNROLL):
                kvec = (k * UNROLL + u) * LANES + iota
                for j in range(BPT):
                    entp = plsc.load_gather(ent_v, [pbase[j] + kvec])
                    wv = plsc.load_gather(w_v, [pbase[j] + kvec])
                    av = plsc.load_gather(add_v, [j * NUM_BOX + kvec])
                    uu = entp + av * wv
                    plsc.store_scatter(ent_v, [pbase[j] + kvec], uu)
                    mx[j] = jnp.maximum(mx[j], jnp.abs(uu))
            return tuple(mx)
        mxv = lax.fori_loop(0, NCHUNK // UNROLL, _upd,
                            tuple(jnp.zeros((LANES,), jnp.float32)
                                  for _ in range(BPT)))
        sc_spl = []
        for j in range(BPT):
            m = jnp.max(mxv[j])
            m_spl = jnp.full((LANES,), jnp.where(m <= 1.0, jnp.float32(1.0), m),
                             jnp.float32)
            sc_spl.append(jnp.full((LANES,), 1.0, jnp.float32) / m_spl)

        def _nrm(k, carry):
            for u in range(UNROLL):
                kvec = (k * UNROLL + u) * LANES + iota
                for j in range(BPT):
                    uu = plsc.load_gather(ent_v, [pbase[j] + kvec])
                    plsc.store_scatter(ent_v, [pbase[j] + kvec], uu * sc_spl[j])
            return carry
        lax.fori_loop(0, NCHUNK // UNROLL, _nrm, 0)

        for j in range(BPT):
            plsc.store_scatter(add_v, [j * NUM_BOX + umv[j]], eps_v,
                               mask=mask_ctx)

    for j in range(BPT):
        c_out.append(pltpu.async_copy(ent_v.at[pl.ds(j * ENT_W, ENT_W)],
                                      out_hbm.at[b0 + j], sem_out.at[j]))
    for c in c_out:
        c.wait()


@functools.cache
def _build():
    mesh = plsc.VectorSubcoreMesh(core_axis_name="c", subcore_axis_name="s")
    return pl.kernel(
        _tdp_body,
        mesh=mesh,
        compiler_params=pltpu.CompilerParams(
            needs_layout_passes=False, use_tc_tiling_on_sc=False),
        out_type=jax.ShapeDtypeStruct((BS, ENT_W), jnp.float32),
        scratch_types=[
            pltpu.VMEM((BPT * ENT_N,), jnp.int32),
            pltpu.VMEM((BPT * ENT_N,), jnp.int32),
            pltpu.VMEM((BPT * LANES,), jnp.int32),
            pltpu.VMEM((BPT * ENT_N, 1), jnp.int32),
            pltpu.VMEM((8, ROW_W), jnp.float32),
            pltpu.VMEM((BPT * ENT_W,), jnp.float32),
            pltpu.VMEM((BPT * ENT_W,), jnp.float32),
            pltpu.VMEM((BPT * NUM_BOX,), jnp.float32),
            pltpu.SemaphoreType.DMA((3,)),
            pltpu.SemaphoreType.DMA((BPT,)),
            pltpu.SemaphoreType.DMA((BPT,)),
            pltpu.SemaphoreType.DMA((8,)),
            pltpu.SemaphoreType.DMA((BPT,)),
        ],
    )


def kernel(traversal_lists, adj_matrices, ent_attn, spo_attn, ctx_idx_adjusted,
           roi_cls, roi_mask, weight_on_children):
    tl = jnp.asarray(traversal_lists, jnp.int32)
    adj0 = jnp.asarray(adj_matrices[:, 0, :], jnp.int32)
    um = jnp.asarray(ctx_idx_adjusted[:, 0, :], jnp.int32)
    um_pad = jnp.concatenate(
        [um, jnp.zeros((BS, LANES - NUM_CTX), jnp.int32)], axis=1)
    spo_flat = jnp.asarray(spo_attn, jnp.float32).reshape(BS * SPO_N, ROW_W)
    ent = jnp.asarray(ent_attn, jnp.float32).reshape(BS, ENT_W)
    w = jnp.asarray(weight_on_children, jnp.float32).reshape(BS, ENT_W)
    out = _build()(tl, adj0, um_pad, ent, spo_flat, w)
    return out.reshape(BS, ENT_N, NUM_BOX)


# speedup vs baseline: 1.1225x; 1.1225x over previous
import functools

import jax
import jax.numpy as jnp
from jax import lax
from jax.experimental import pallas as pl
from jax.experimental.pallas import tpu as pltpu
from jax.experimental.pallas import tpu_sc as plsc

BS = 64
ENT_N = 16
SPO_N = 16
NUM_BOX = 1024
NUM_CTX = 5
LANES = 16
NCHUNK = NUM_BOX // LANES
UNROLL = 2
NW = 32
BPT = BS // NW
ROW_W = NUM_BOX * NUM_CTX
EPS = 1e-6


def _tdp_body(tl_hbm, adj_hbm, um_hbm, ent_hbm, spo_hbm, w_hbm, out_hbm,
              tl_v, adj_v, um_v, rowid_v, ring_v, ent_v, w_v, add_v,
              sem_meta, sem_ent, sem_w, sem_ring, sem_out):
    cid = lax.axis_index("c")
    sid = lax.axis_index("s")
    wid = sid * 2 + cid
    b0 = wid * BPT

    iota = lax.iota(jnp.int32, LANES)
    zeros_i = jnp.zeros((LANES,), jnp.int32)
    mask_ctx = iota < NUM_CTX
    eps_v = jnp.full((LANES,), EPS, jnp.float32)

    c_meta = [
        pltpu.async_copy(tl_hbm.at[pl.ds(b0, BPT)], tl_v, sem_meta.at[0]),
        pltpu.async_copy(adj_hbm.at[pl.ds(b0, BPT)], adj_v, sem_meta.at[1]),
        pltpu.async_copy(um_hbm.at[pl.ds(b0, BPT)], um_v, sem_meta.at[2]),
    ]
    c_ent = [pltpu.async_copy(ent_hbm.at[b0 + j], ent_v.at[j], sem_ent.at[j])
             for j in range(BPT)]
    c_w = [pltpu.async_copy(w_hbm.at[b0 + j], w_v.at[j], sem_w.at[j])
           for j in range(BPT)]
    for c in c_meta:
        c.wait()

    tlv = []
    for j in range(BPT):
        jsplat = jnp.full((LANES,), j, jnp.int32)
        tlv.append(tl_v[j])
        e0 = plsc.load_gather(adj_v, [jsplat, tlv[j]])
        rid = e0 + (b0 + j) * ENT_N
        plsc.store_scatter(rowid_v, [j * ENT_N + iota, zeros_i], rid)

    def _init(k, carry):
        plsc.store_scatter(add_v, [k * LANES + iota], eps_v)
        return carry
    lax.fori_loop(0, BPT * NCHUNK, _init, 0)

    def _fetch(j, it):
        s = 2 * (it % 4) + j
        return pltpu.async_copy(
            spo_hbm.at[rowid_v.at[j * ENT_N + it]], ring_v.at[s],
            sem_ring.at[s])

    handles = {}
    for pit in range(3):
        for j in range(BPT):
            handles[(j, pit)] = _fetch(j, pit)

    for j in range(BPT):
        c_ent[j].wait()
        c_w[j].wait()
    umv = [um_v[j] for j in range(BPT)]

    c_out = []
    for it in range(ENT_N):
        if it + 3 < ENT_N:
            for j in range(BPT):
                handles[(j, it + 3)] = _fetch(j, it + 3)
        for j in range(BPT):
            handles[(j, it)].wait()

        p_spl = [jnp.full((LANES,), tlv[j][it], jnp.int32) for j in range(BPT)]
        jv = [jnp.full((LANES,), j, jnp.int32) for j in range(BPT)]
        rslot = [jnp.full((LANES,), 2 * (it % 4) + j, jnp.int32)
                 for j in range(BPT)]

        def _dot(k, accs):
            accs = list(accs)
            for u in range(UNROLL):
                kvec = (k * UNROLL + u) * LANES + iota
                base5 = kvec * NUM_CTX
                for j in range(BPT):
                    e0c = plsc.load_gather(ent_v, [jv[j], zeros_i, kvec])
                    for c in range(NUM_CTX):
                        accs[j * NUM_CTX + c] = accs[j * NUM_CTX + c] + e0c * \
                            plsc.load_gather(ring_v, [rslot[j], zeros_i, base5 + c])
            return tuple(accs)
        accs = lax.fori_loop(
            0, NCHUNK // UNROLL, _dot,
            tuple(jnp.zeros((LANES,), jnp.float32) for _ in range(BPT * NUM_CTX)))
        for j in range(BPT):
            tvec = jnp.zeros((LANES,), jnp.float32)
            for c in range(NUM_CTX):
                tvec = jnp.where(iota == c,
                                 jnp.sum(accs[j * NUM_CTX + c]) + jnp.float32(EPS),
                                 tvec)
            plsc.store_scatter(add_v, [j * NUM_BOX + umv[j]], tvec, mask=mask_ctx)

        def _upd(k, mx):
            mx = list(mx)
            for u in range(UNROLL):
                kvec = (k * UNROLL + u) * LANES + iota
                for j in range(BPT):
                    entp = plsc.load_gather(ent_v, [jv[j], p_spl[j], kvec])
                    wv = plsc.load_gather(w_v, [jv[j], p_spl[j], kvec])
                    av = plsc.load_gather(add_v, [j * NUM_BOX + kvec])
                    uu = entp + av * wv
                    plsc.store_scatter(ent_v, [jv[j], p_spl[j], kvec], uu)
                    mx[j] = jnp.maximum(mx[j], jnp.abs(uu))
            return tuple(mx)
        mxv = lax.fori_loop(0, NCHUNK // UNROLL, _upd,
                            tuple(jnp.zeros((LANES,), jnp.float32)
                                  for _ in range(BPT)))
        sc_spl = []
        for j in range(BPT):
            m = jnp.max(mxv[j])
            m_spl = jnp.full((LANES,), jnp.where(m <= 1.0, jnp.float32(1.0), m),
                             jnp.float32)
            sc_spl.append(jnp.full((LANES,), 1.0, jnp.float32) / m_spl)

        def _nrm(k, carry):
            for u in range(UNROLL):
                kvec = (k * UNROLL + u) * LANES + iota
                for j in range(BPT):
                    uu = plsc.load_gather(ent_v, [jv[j], p_spl[j], kvec])
                    plsc.store_scatter(ent_v, [jv[j], p_spl[j], kvec],
                                       uu * sc_spl[j])
            return carry
        lax.fori_loop(0, NCHUNK // UNROLL, _nrm, 0)

        for j in range(BPT):
            plsc.store_scatter(add_v, [j * NUM_BOX + umv[j]], eps_v,
                               mask=mask_ctx)

    for j in range(BPT):
        c_out.append(pltpu.async_copy(ent_v.at[j], out_hbm.at[b0 + j],
                                      sem_out.at[j]))
    for c in c_out:
        c.wait()


@functools.cache
def _build():
    mesh = plsc.VectorSubcoreMesh(core_axis_name="c", subcore_axis_name="s")
    return pl.kernel(
        _tdp_body,
        mesh=mesh,
        compiler_params=pltpu.CompilerParams(needs_layout_passes=False),
        out_type=jax.ShapeDtypeStruct((BS, ENT_N, NUM_BOX), jnp.float32),
        scratch_types=[
            pltpu.VMEM((BPT, ENT_N), jnp.int32),
            pltpu.VMEM((BPT, ENT_N), jnp.int32),
            pltpu.VMEM((BPT, LANES), jnp.int32),
            pltpu.VMEM((BPT * ENT_N, 1), jnp.int32),
            pltpu.VMEM((8, 1, ROW_W), jnp.float32),
            pltpu.VMEM((BPT, ENT_N, NUM_BOX), jnp.float32),
            pltpu.VMEM((BPT, ENT_N, NUM_BOX), jnp.float32),
            pltpu.VMEM((BPT * NUM_BOX,), jnp.float32),
            pltpu.SemaphoreType.DMA((3,)),
            pltpu.SemaphoreType.DMA((BPT,)),
            pltpu.SemaphoreType.DMA((BPT,)),
            pltpu.SemaphoreType.DMA((8,)),
            pltpu.SemaphoreType.DMA((BPT,)),
        ],
    )


def kernel(traversal_lists, adj_matrices, ent_attn, spo_attn, ctx_idx_adjusted,
           roi_cls, roi_mask, weight_on_children):
    tl = jnp.asarray(traversal_lists, jnp.int32)
    adj0 = jnp.asarray(adj_matrices[:, 0, :], jnp.int32)
    um = jnp.asarray(ctx_idx_adjusted[:, 0, :], jnp.int32)
    um_pad = jnp.concatenate(
        [um, jnp.zeros((BS, LANES - NUM_CTX), jnp.int32)], axis=1)
    spo_flat = jnp.asarray(spo_attn, jnp.float32).reshape(BS * SPO_N, ROW_W)
    return _build()(tl, adj0, um_pad, ent_attn, spo_flat, weight_on_children)


# speedup vs baseline: 2.6090x; 2.3243x over previous
import functools

import jax
import jax.numpy as jnp
from jax import lax
from jax.experimental import pallas as pl
from jax.experimental.pallas import tpu as pltpu
from jax.experimental.pallas import tpu_sc as plsc

BS = 64
ENT_N = 16
SPO_N = 16
NUM_BOX = 1024
NUM_CTX = 5
LANES = 16
NCHUNK = NUM_BOX // LANES
UNROLL = 2
NW = 32
BPT = BS // NW
ROW_W = NUM_BOX * NUM_CTX
EPS = 1e-6


def _tdp_body(tl_hbm, adj_hbm, um_hbm, ent_hbm, spo_hbm, w_hbm, out_hbm,
              tl_v, adj_v, um_v, ring_v, ent_v, w_v, add_v,
              sem_meta, sem_ent, sem_w, sem_ring, sem_out):
    cid = lax.axis_index("c")
    sid = lax.axis_index("s")
    wid = sid * 2 + cid
    b0 = wid * BPT

    iota = lax.iota(jnp.int32, LANES)
    zeros_i = jnp.zeros((LANES,), jnp.int32)
    mask_ctx = iota < NUM_CTX
    eps_v = jnp.full((LANES,), EPS, jnp.float32)

    c_meta = [
        pltpu.async_copy(tl_hbm.at[pl.ds(b0, BPT)], tl_v, sem_meta.at[0]),
        pltpu.async_copy(adj_hbm.at[pl.ds(b0, BPT)], adj_v, sem_meta.at[1]),
        pltpu.async_copy(um_hbm.at[pl.ds(b0, BPT)], um_v, sem_meta.at[2]),
    ]
    c_ent = [pltpu.async_copy(ent_hbm.at[b0 + j], ent_v.at[j], sem_ent.at[j])
             for j in range(BPT)]
    c_w = [pltpu.async_copy(w_hbm.at[b0 + j], w_v.at[j], sem_w.at[j])
           for j in range(BPT)]
    for c in c_meta:
        c.wait()

    tlv = []
    e0v = []
    for j in range(BPT):
        jsplat = jnp.full((LANES,), j, jnp.int32)
        tlv.append(tl_v[j])
        e0v.append(plsc.load_gather(adj_v, [jsplat, tlv[j]]))

    def _init(k, carry):
        plsc.store_scatter(add_v, [k * LANES + iota], eps_v)
        return carry
    lax.fori_loop(0, BPT * NCHUNK, _init, 0)

    def _fetch(j, it):
        s = 2 * (it % 3) + j
        rid = (b0 + j) * ENT_N + e0v[j][it]
        return pltpu.async_copy(spo_hbm.at[rid], ring_v.at[s], sem_ring.at[s])

    handles = {}
    for pit in range(2):
        for j in range(BPT):
            handles[(j, pit)] = _fetch(j, pit)

    for j in range(BPT):
        c_ent[j].wait()
        c_w[j].wait()
    umv = [um_v[j] for j in range(BPT)]

    c_out = []
    for it in range(ENT_N):
        if it + 2 < ENT_N:
            for j in range(BPT):
                handles[(j, it + 2)] = _fetch(j, it + 2)
        for j in range(BPT):
            handles[(j, it)].wait()

        p_spl = [jnp.full((LANES,), tlv[j][it], jnp.int32) for j in range(BPT)]
        jv = [jnp.full((LANES,), j, jnp.int32) for j in range(BPT)]
        rslot = [jnp.full((LANES,), 2 * (it % 3) + j, jnp.int32)
                 for j in range(BPT)]

        def _dot(k, accs):
            accs = list(accs)
            for u in range(UNROLL):
                kvec = (k * UNROLL + u) * LANES + iota
                for j in range(BPT):
                    e0c = plsc.load_gather(ent_v, [jv[j], zeros_i, kvec])
                    for c in range(NUM_CTX):
                        accs[j * NUM_CTX + c] = accs[j * NUM_CTX + c] + e0c * \
                            plsc.load_gather(
                                ring_v,
                                [rslot[j], jnp.full((LANES,), c, jnp.int32), kvec])
            return tuple(accs)
        accs = lax.fori_loop(
            0, NCHUNK // UNROLL, _dot,
            tuple(jnp.zeros((LANES,), jnp.float32) for _ in range(BPT * NUM_CTX)))
        for j in range(BPT):
            tvec = jnp.zeros((LANES,), jnp.float32)
            for c in range(NUM_CTX):
                tvec = jnp.where(iota == c,
                                 jnp.sum(accs[j * NUM_CTX + c]) + jnp.float32(EPS),
                                 tvec)
            plsc.store_scatter(add_v, [j * NUM_BOX + umv[j]], tvec, mask=mask_ctx)

        def _upd(k, mx):
            mx = list(mx)
            for u in range(UNROLL):
                kvec = (k * UNROLL + u) * LANES + iota
                for j in range(BPT):
                    entp = plsc.load_gather(ent_v, [jv[j], p_spl[j], kvec])
                    wv = plsc.load_gather(w_v, [jv[j], p_spl[j], kvec])
                    av = plsc.load_gather(add_v, [j * NUM_BOX + kvec])
                    uu = entp + av * wv
                    plsc.store_scatter(ent_v, [jv[j], p_spl[j], kvec], uu)
                    mx[j] = jnp.maximum(mx[j], jnp.abs(uu))
            return tuple(mx)
        mxv = lax.fori_loop(0, NCHUNK // UNROLL, _upd,
                            tuple(jnp.zeros((LANES,), jnp.float32)
                                  for _ in range(BPT)))
        sc_spl = []
        for j in range(BPT):
            m = jnp.max(mxv[j])
            m_spl = jnp.full((LANES,), jnp.where(m <= 1.0, jnp.float32(1.0), m),
                             jnp.float32)
            sc_spl.append(jnp.full((LANES,), 1.0, jnp.float32) / m_spl)

        def _nrm(k, carry):
            for u in range(UNROLL):
                kvec = (k * UNROLL + u) * LANES + iota
                for j in range(BPT):
                    uu = plsc.load_gather(ent_v, [jv[j], p_spl[j], kvec])
                    plsc.store_scatter(ent_v, [jv[j], p_spl[j], kvec],
                                       uu * sc_spl[j])
            return carry
        lax.fori_loop(0, NCHUNK // UNROLL, _nrm, 0)

        for j in range(BPT):
            plsc.store_scatter(add_v, [j * NUM_BOX + umv[j]], eps_v,
                               mask=mask_ctx)

    for j in range(BPT):
        c_out.append(pltpu.async_copy(ent_v.at[j], out_hbm.at[b0 + j],
                                      sem_out.at[j]))
    for c in c_out:
        c.wait()


@functools.cache
def _build():
    mesh = plsc.VectorSubcoreMesh(core_axis_name="c", subcore_axis_name="s")
    return pl.kernel(
        _tdp_body,
        mesh=mesh,
        compiler_params=pltpu.CompilerParams(needs_layout_passes=False),
        out_type=jax.ShapeDtypeStruct((BS, ENT_N, NUM_BOX), jnp.float32),
        scratch_types=[
            pltpu.VMEM((BPT, ENT_N), jnp.int32),
            pltpu.VMEM((BPT, ENT_N), jnp.int32),
            pltpu.VMEM((BPT, LANES), jnp.int32),
            pltpu.VMEM((6, 8, NUM_BOX), jnp.float32),
            pltpu.VMEM((BPT, ENT_N, NUM_BOX), jnp.float32),
            pltpu.VMEM((BPT, ENT_N, NUM_BOX), jnp.float32),
            pltpu.VMEM((BPT * NUM_BOX,), jnp.float32),
            pltpu.SemaphoreType.DMA((3,)),
            pltpu.SemaphoreType.DMA((BPT,)),
            pltpu.SemaphoreType.DMA((BPT,)),
            pltpu.SemaphoreType.DMA((6,)),
            pltpu.SemaphoreType.DMA((BPT,)),
        ],
    )


def kernel(traversal_lists, adj_matrices, ent_attn, spo_attn, ctx_idx_adjusted,
           roi_cls, roi_mask, weight_on_children):
    tl = jnp.asarray(traversal_lists, jnp.int32)
    adj0 = jnp.asarray(adj_matrices[:, 0, :], jnp.int32)
    um = jnp.asarray(ctx_idx_adjusted[:, 0, :], jnp.int32)
    um_pad = jnp.concatenate(
        [um, jnp.zeros((BS, LANES - NUM_CTX), jnp.int32)], axis=1)
    spo_t = jnp.pad(
        jnp.transpose(jnp.asarray(spo_attn, jnp.float32), (0, 1, 3, 2)),
        ((0, 0), (0, 0), (0, 8 - NUM_CTX), (0, 0))).reshape(BS * SPO_N, 8, NUM_BOX)
    return _build()(tl, adj0, um_pad, ent_attn, spo_t, weight_on_children)
